# Initial kernel scaffold; baseline (speedup 1.0000x reference)
import functools

import jax
import jax.numpy as jnp
from jax import lax
from jax.experimental import pallas as pl
from jax.experimental.pallas import tpu as pltpu
from jax.experimental.pallas import tpu_sc as plsc

_N = 10000
_E = 320000
_IN_C = 128
_HID = 128
_OUT_C = 64

_NC = 2
_NS = 16
_NW = _NC * _NS
_EPW = _E // _NW
_CHUNK = 80
_NCHUNK = _EPW // _CHUNK
_RPT = _N // _NS
_ZROWS = 125


def _sc_body(x_hbm, src_hbm, dst_hbm, agg_hbm, degp_hbm,
             src_v, dst_v, rows_v, deg_v, zbuf_v, agg_sh, sem):
    cid = lax.axis_index("c")
    sid = lax.axis_index("s")
    wid = sid * _NC + cid

    zero16 = jnp.zeros((16,), jnp.float32)

    @pl.loop(0, _N // 16)
    def _(k):
        deg_v[pl.ds(k * 16, 16)] = zero16

    @pl.loop(0, _ZROWS * _IN_C // 16)
    def _(k):
        zbuf_v[k // 8, pl.ds((k % 8) * 16, 16)] = zero16

    for t in range(_RPT // _ZROWS):
        pltpu.sync_copy(zbuf_v, agg_sh.at[pl.ds(sid * _RPT + t * _ZROWS, _ZROWS)])
    plsc.subcore_barrier()

    pltpu.sync_copy(src_hbm.at[wid], src_v)
    pltpu.sync_copy(dst_hbm.at[wid], dst_v)

    ones16 = jnp.ones((16,), jnp.float32)

    @pl.loop(0, _NCHUNK)
    def _(c):
        pltpu.async_copy(x_hbm.at[src_v.at[c]], rows_v, sem).wait()
        pltpu.sync_copy(rows_v, agg_sh.at[dst_v.at[c]], add=True)
        for j in range(_CHUNK // 16):
            idx = dst_v[c, pl.ds(j * 16, 16)]
            plsc.addupdate_scatter(deg_v, [idx], ones16)

    plsc.subcore_barrier()

    pltpu.sync_copy(agg_sh.at[pl.ds(sid * _RPT, _RPT)],
                    agg_hbm.at[cid, pl.ds(sid * _RPT, _RPT)])
    pltpu.sync_copy(deg_v, degp_hbm.at[wid])


_sc_agg = functools.partial(
    pl.kernel,
    out_type=(
        jax.ShapeDtypeStruct((_NC, _N, _IN_C), jnp.float32),
        jax.ShapeDtypeStruct((_NW, _N), jnp.float32),
    ),
    mesh=plsc.VectorSubcoreMesh(core_axis_name="c", subcore_axis_name="s",
                                num_cores=_NC, num_subcores=_NS),
    scratch_types=[
        pltpu.VMEM((_NCHUNK, _CHUNK), jnp.int32),
        pltpu.VMEM((_NCHUNK, _CHUNK), jnp.int32),
        pltpu.VMEM((_CHUNK, _IN_C), jnp.float32),
        pltpu.VMEM((_N,), jnp.float32),
        pltpu.VMEM((_ZROWS, _IN_C), jnp.float32),
        pltpu.VMEM_SHARED((_N, _IN_C), jnp.float32),
        pltpu.SemaphoreType.DMA,
    ],
)(_sc_body)


def _tc_body(x_ref, agg_ref, degp_ref, wl_ref, bl_ref, wr_ref,
             wfc1_ref, bfc1_ref, wout_ref, bout_ref, out_ref):
    deg = jnp.sum(degp_ref[...], axis=0)
    agg = agg_ref[0] + agg_ref[1]
    mean = agg / jnp.maximum(deg, 1.0)[:, None]
    h = (jnp.dot(mean, wl_ref[...], preferred_element_type=jnp.float32)
         + bl_ref[...]
         + jnp.dot(x_ref[...], wr_ref[...], preferred_element_type=jnp.float32))
    h = jnp.maximum(h, 0.0)
    h = jnp.maximum(
        jnp.dot(h, wfc1_ref[...], preferred_element_type=jnp.float32)
        + bfc1_ref[...], 0.0)
    logits = (jnp.dot(h, wout_ref[...], preferred_element_type=jnp.float32)
              + bout_ref[...])
    m = jnp.max(logits, axis=1, keepdims=True)
    s = jnp.sum(jnp.exp(logits - m), axis=1, keepdims=True)
    out_ref[...] = logits - m - jnp.log(s)


_ROWS_BLK = 1000


def _tc_head(x, agg2, degp, wl_t, bl, wr_t, wfc1_t, bfc1, wout_t, bout):
    grid = (_N // _ROWS_BLK,)
    full = lambda i: (0, 0)
    return pl.pallas_call(
        _tc_body,
        grid=grid,
        in_specs=[
            pl.BlockSpec((_ROWS_BLK, _IN_C), lambda i: (i, 0)),
            pl.BlockSpec((_NC, _ROWS_BLK, _IN_C), lambda i: (0, i, 0)),
            pl.BlockSpec((_NW, _ROWS_BLK), lambda i: (0, i)),
            pl.BlockSpec((_IN_C, _HID), full),
            pl.BlockSpec((1, _HID), full),
            pl.BlockSpec((_IN_C, _HID), full),
            pl.BlockSpec((_HID, _HID), full),
            pl.BlockSpec((1, _HID), full),
            pl.BlockSpec((_HID, _OUT_C), full),
            pl.BlockSpec((1, _OUT_C), full),
        ],
        out_specs=pl.BlockSpec((_ROWS_BLK, _OUT_C), lambda i: (i, 0)),
        out_shape=jax.ShapeDtypeStruct((_N, _OUT_C), jnp.float32),
    )(x, agg2, degp, wl_t, bl, wr_t, wfc1_t, bfc1, wout_t, bout)


def kernel(x, edge_index, W_l, b_l, W_r, W_fc1, b_fc1, W_out, b_out):
    src = edge_index[0].reshape(_NW, _NCHUNK, _CHUNK)
    dst = edge_index[1].reshape(_NW, _NCHUNK, _CHUNK)
    agg2, degp = _sc_agg(x, src, dst)
    return _tc_head(x, agg2, degp,
                    W_l.T, b_l.reshape(1, _HID),
                    W_r.T, W_fc1.T, b_fc1.reshape(1, _HID),
                    W_out.T, b_out.reshape(1, _OUT_C))


# speedup vs baseline: 4.8921x; 4.8921x over previous
import functools

import jax
import jax.numpy as jnp
from jax import lax
from jax.experimental import pallas as pl
from jax.experimental.pallas import tpu as pltpu
from jax.experimental.pallas import tpu_sc as plsc

_N = 10000
_E = 320000
_IN_C = 128
_HID = 128
_OUT_C = 64

_NC = 2
_NS = 16
_NW = _NC * _NS
_EPT = _E // _NS
_CHUNK = 80
_NCHUNK = _EPT // _CHUNK
_HALF = _N // _NC
_ACC = 5248
_TRASH = 5240
_RPT = _ACC // _NS
_ZROWS = 41


def _sc_body(x_hbm, src_hbm, dst_hbm, agg_hbm, degp_hbm,
             src_v, dst_v, rows_v, deg_v, zbuf_v, agg_sh, sem):
    cid = lax.axis_index("c")
    sid = lax.axis_index("s")
    wid = sid * _NC + cid

    zero16 = jnp.zeros((16,), jnp.float32)

    @pl.loop(0, _ACC // 16)
    def _(k):
        deg_v[pl.ds(k * 16, 16)] = zero16

    @pl.loop(0, _ZROWS * _IN_C // 16)
    def _(k):
        zbuf_v[k // 8, pl.ds((k % 8) * 16, 16)] = zero16

    for t in range(_RPT // _ZROWS):
        pltpu.sync_copy(zbuf_v, agg_sh.at[pl.ds(sid * _RPT + t * _ZROWS, _ZROWS)])
    plsc.subcore_barrier()

    pltpu.sync_copy(src_hbm.at[sid], src_v)
    pltpu.sync_copy(dst_hbm.at[sid], dst_v)

    ones16 = jnp.ones((16,), jnp.float32)
    base = cid * _HALF

    @pl.loop(0, _EPT // 16)
    def _(k):
        r = k // (_CHUNK // 16)
        j = k % (_CHUNK // 16)
        d = dst_v[r, pl.ds(j * 16, 16)]
        dl = d - base
        valid = (dl >= 0) & (dl < _HALF)
        dmap = jnp.where(valid, dl, _TRASH)
        dst_v[r, pl.ds(j * 16, 16)] = dmap
        plsc.addupdate_scatter(deg_v, [dmap], ones16)

    @pl.loop(0, _NCHUNK)
    def _(c):
        pltpu.async_copy(x_hbm.at[src_v.at[c]], rows_v, sem).wait()
        pltpu.sync_copy(rows_v, agg_sh.at[dst_v.at[c]], add=True)

    plsc.subcore_barrier()

    pltpu.sync_copy(agg_sh.at[pl.ds(sid * _RPT, _RPT)],
                    agg_hbm.at[cid, pl.ds(sid * _RPT, _RPT)])
    pltpu.sync_copy(deg_v, degp_hbm.at[pl.ds(wid * _ACC, _ACC)])


_sc_agg = functools.partial(
    pl.kernel,
    out_type=(
        jax.ShapeDtypeStruct((_NC, _ACC, _IN_C), jnp.float32),
        jax.ShapeDtypeStruct((_NW * _ACC,), jnp.float32),
    ),
    mesh=plsc.VectorSubcoreMesh(core_axis_name="c", subcore_axis_name="s",
                                num_cores=_NC, num_subcores=_NS),
    scratch_types=[
        pltpu.VMEM((_NCHUNK, _CHUNK), jnp.int32),
        pltpu.VMEM((_NCHUNK, _CHUNK), jnp.int32),
        pltpu.VMEM((_CHUNK, _IN_C), jnp.float32),
        pltpu.VMEM((_ACC,), jnp.float32),
        pltpu.VMEM((_ZROWS, _IN_C), jnp.float32),
        pltpu.VMEM_SHARED((_ACC, _IN_C), jnp.float32),
        pltpu.SemaphoreType.DMA,
    ],
    compiler_params=pltpu.CompilerParams(needs_layout_passes=False),
)(_sc_body)


def _tc_body(x_ref, agg_ref, degt_ref, wl_ref, bl_ref, wr_ref,
             wfc1_ref, bfc1_ref, wout_ref, bout_ref, out_ref):
    deg = jnp.sum(degt_ref[0], axis=1)
    mean = agg_ref[0] / jnp.maximum(deg, 1.0)[:, None]
    h = (jnp.dot(mean, wl_ref[...], preferred_element_type=jnp.float32)
         + bl_ref[...]
         + jnp.dot(x_ref[...], wr_ref[...], preferred_element_type=jnp.float32))
    h = jnp.maximum(h, 0.0)
    h = jnp.maximum(
        jnp.dot(h, wfc1_ref[...], preferred_element_type=jnp.float32)
        + bfc1_ref[...], 0.0)
    logits = (jnp.dot(h, wout_ref[...], preferred_element_type=jnp.float32)
              + bout_ref[...])
    m = jnp.max(logits, axis=1, keepdims=True)
    s = jnp.sum(jnp.exp(logits - m), axis=1, keepdims=True)
    out_ref[...] = logits - m - jnp.log(s)


_ROWS_BLK = 1000
_BLK_PER_HALF = _HALF // _ROWS_BLK


def _tc_head(x, agg2, degt, wl_t, bl, wr_t, wfc1_t, bfc1, wout_t, bout):
    grid = (_N // _ROWS_BLK,)
    full = lambda i: (0, 0)
    return pl.pallas_call(
        _tc_body,
        grid=grid,
        in_specs=[
            pl.BlockSpec((_ROWS_BLK, _IN_C), lambda i: (i, 0)),
            pl.BlockSpec((1, _ROWS_BLK, _IN_C),
                         lambda i: (i // _BLK_PER_HALF, i % _BLK_PER_HALF, 0)),
            pl.BlockSpec((1, _ROWS_BLK, _NS),
                         lambda i: (i // _BLK_PER_HALF, i % _BLK_PER_HALF, 0)),
            pl.BlockSpec((_IN_C, _HID), full),
            pl.BlockSpec((1, _HID), full),
            pl.BlockSpec((_IN_C, _HID), full),
            pl.BlockSpec((_HID, _HID), full),
            pl.BlockSpec((1, _HID), full),
            pl.BlockSpec((_HID, _OUT_C), full),
            pl.BlockSpec((1, _OUT_C), full),
        ],
        out_specs=pl.BlockSpec((_ROWS_BLK, _OUT_C), lambda i: (i, 0)),
        out_shape=jax.ShapeDtypeStruct((_N, _OUT_C), jnp.float32),
    )(x, agg2, degt, wl_t, bl, wr_t, wfc1_t, bfc1, wout_t, bout)


def kernel(x, edge_index, W_l, b_l, W_r, W_fc1, b_fc1, W_out, b_out):
    src = edge_index[0].reshape(_NS, _NCHUNK, _CHUNK)
    dst = edge_index[1].reshape(_NS, _NCHUNK, _CHUNK)
    agg2, degp = _sc_agg(x, src, dst)
    degt = degp.reshape(_NS, _NC, _ACC).transpose(1, 2, 0)
    return _tc_head(x, agg2, degt,
                    W_l.T, b_l.reshape(1, _HID),
                    W_r.T, W_fc1.T, b_fc1.reshape(1, _HID),
                    W_out.T, b_out.reshape(1, _OUT_C))
